# baseline (speedup 1.0000x reference)
import functools

import jax
import jax.numpy as jnp
from jax import lax
from jax.experimental import pallas as pl
from jax.experimental.pallas import tpu as pltpu
from jax.experimental.pallas import tpu_sc as plsc

VOCAB = 100000
DIM = 100
SEQ = 200
BATCH = 4096

NC = 2
NS = 16
NW = NC * NS
CH = 100
CHP = 104
NCHUNK = BATCH * SEQ // CH
CPW = NCHUNK // NW

_mesh = plsc.VectorSubcoreMesh(core_axis_name="c", subcore_axis_name="s")


@functools.partial(
    pl.kernel,
    mesh=_mesh,
    out_type=jax.ShapeDtypeStruct((BATCH, SEQ, DIM), jnp.float32),
    scratch_types=[
        pltpu.VMEM((CHP,), jnp.int32),
        pltpu.VMEM((CHP, DIM), jnp.float32),
        pltpu.SemaphoreType.DMA,
    ],
    compiler_params=pltpu.CompilerParams(use_tc_tiling_on_sc=False),
)
def _gather_kernel(ids_hbm, table_hbm, out_hbm, idx_v, rows_v, sem):
    w = lax.axis_index("s") * NC + lax.axis_index("c")

    def body(c, carry):
        g = w * CPW + c
        b = g // 2
        s0 = (g % 2) * CH
        pltpu.sync_copy(ids_hbm.at[pl.ds(g * CHP, CHP)], idx_v)
        pltpu.async_copy(table_hbm.at[idx_v], rows_v, sem).wait()
        pltpu.sync_copy(rows_v.at[pl.ds(0, CH)], out_hbm.at[b, pl.ds(s0, CH)])
        return carry

    lax.fori_loop(0, CPW, body, 0)


def kernel(input_ids, table):
    ids_t = jnp.transpose(input_ids, (1, 0)).reshape(NCHUNK, CH)
    ids_pad = jnp.pad(ids_t.astype(jnp.int32), ((0, 0), (0, CHP - CH)))
    return _gather_kernel(ids_pad.reshape(NCHUNK * CHP), table)


# speedup vs baseline: 1.7662x; 1.7662x over previous
import functools

import jax
import jax.numpy as jnp
from jax import lax
from jax.experimental import pallas as pl
from jax.experimental.pallas import tpu as pltpu
from jax.experimental.pallas import tpu_sc as plsc

VOCAB = 100000
DIM = 100
SEQ = 200
BATCH = 4096

NC = 2
NS = 16
NW = NC * NS
ROWS = SEQ * BATCH
RPW = ROWS // NW
CH = 128
NCH = RPW // CH

_mesh = plsc.VectorSubcoreMesh(core_axis_name="c", subcore_axis_name="s")


@functools.partial(
    pl.kernel,
    mesh=_mesh,
    out_type=jax.ShapeDtypeStruct((ROWS, DIM), jnp.float32),
    scratch_types=[
        pltpu.VMEM((CH,), jnp.int32),
        pltpu.VMEM((CH, DIM), jnp.float32),
        pltpu.SemaphoreType.DMA,
    ],
    compiler_params=pltpu.CompilerParams(use_tc_tiling_on_sc=False),
)
def _gather_kernel(ids_hbm, table_hbm, out_hbm, idx_v, rows_v, sem):
    w = lax.axis_index("s") * NC + lax.axis_index("c")
    base = w * RPW

    def body(j, carry):
        off = base + j * CH
        pltpu.sync_copy(ids_hbm.at[pl.ds(off, CH)], idx_v)
        pltpu.async_copy(table_hbm.at[idx_v], rows_v, sem).wait()
        pltpu.sync_copy(rows_v, out_hbm.at[pl.ds(off, CH)])
        return carry

    lax.fori_loop(0, NCH, body, 0)


def kernel(input_ids, table):
    ids_t = jnp.transpose(input_ids, (1, 0)).reshape(ROWS)
    out = _gather_kernel(ids_t.astype(jnp.int32), table)
    out = jax.lax.optimization_barrier(out)
    return out.reshape(BATCH, SEQ, DIM)


# speedup vs baseline: 2.0950x; 1.1862x over previous
import functools

import jax
import jax.numpy as jnp
from jax import lax
from jax.experimental import pallas as pl
from jax.experimental.pallas import tpu as pltpu
from jax.experimental.pallas import tpu_sc as plsc

VOCAB = 100000
DIM = 100
SEQ = 200
BATCH = 4096

NC = 2
NS = 16
NW = NC * NS
ROWS = SEQ * BATCH
RPW = ROWS // NW
CH = 128
NCH = RPW // CH
NSLOT = 4

_mesh = plsc.VectorSubcoreMesh(core_axis_name="c", subcore_axis_name="s")


@functools.partial(
    pl.kernel,
    mesh=_mesh,
    out_type=jax.ShapeDtypeStruct((ROWS, DIM), jnp.float32),
    scratch_types=[
        pltpu.VMEM((CH,), jnp.int32),
        pltpu.VMEM((CH,), jnp.int32),
        pltpu.VMEM((CH,), jnp.int32),
        pltpu.VMEM((CH,), jnp.int32),
        pltpu.VMEM((NSLOT, CH, DIM), jnp.float32),
        pltpu.SemaphoreType.DMA,
        pltpu.SemaphoreType.DMA,
        pltpu.SemaphoreType.DMA,
        pltpu.SemaphoreType.DMA,
        pltpu.SemaphoreType.DMA,
        pltpu.SemaphoreType.DMA,
        pltpu.SemaphoreType.DMA,
        pltpu.SemaphoreType.DMA,
    ],
    compiler_params=pltpu.CompilerParams(use_tc_tiling_on_sc=False),
)
def _gather_kernel(ids_hbm, table_hbm, out_hbm,
                   i0, i1, i2, i3, rows_v,
                   g0, g1, g2, g3, s0, s1, s2, s3):
    idx = (i0, i1, i2, i3)
    sem_g = (g0, g1, g2, g3)
    sem_s = (s0, s1, s2, s3)
    w = lax.axis_index("s") * NC + lax.axis_index("c")
    base = w * RPW

    def load_idx(j, p):
        pltpu.sync_copy(ids_hbm.at[pl.ds(base + j * CH, CH)], idx[p])

    def gather(j, p):
        return pltpu.make_async_copy(
            table_hbm.at[idx[p]], rows_v.at[p], sem_g[p]
        )

    def store(j, p):
        return pltpu.make_async_copy(
            rows_v.at[p], out_hbm.at[pl.ds(base + j * CH, CH)], sem_s[p]
        )

    for p in range(2):
        load_idx(p, p)
        gather(p, p).start()

    def body(t, carry):
        for p in range(NSLOT):
            j = NSLOT * t + p
            gather(j, p).wait()
            store(j, p).start()
            q = (p + 2) % NSLOT
            jn = j + 2

            @pl.when(jn < NCH)
            def _prefetch():
                @pl.when(j >= 2)
                def _drain_store():
                    store(j - 2, q).wait()

                load_idx(jn, q)
                gather(jn, q).start()

        return carry

    lax.fori_loop(0, NCH // NSLOT, body, 0)

    store(NCH - 2, (NCH - 2) % NSLOT).wait()
    store(NCH - 1, (NCH - 1) % NSLOT).wait()


def kernel(input_ids, table):
    ids_t = jnp.transpose(input_ids, (1, 0)).reshape(ROWS)
    out = _gather_kernel(ids_t.astype(jnp.int32), table)
    return out.reshape(BATCH, SEQ, DIM)


# speedup vs baseline: 2.0950x; 1.0000x over previous
import functools

import jax
import jax.numpy as jnp
from jax import lax
from jax.experimental import pallas as pl
from jax.experimental.pallas import tpu as pltpu
from jax.experimental.pallas import tpu_sc as plsc

VOCAB = 100000
DIM = 100
DPAD = 128
SEQ = 200
BATCH = 4096

NC = 2
NS = 16
NW = NC * NS
ROWS = SEQ * BATCH
RPW = ROWS // NW
CH = 128
NCH = RPW // CH
NSLOT = 4

_mesh = plsc.VectorSubcoreMesh(core_axis_name="c", subcore_axis_name="s")


@functools.partial(
    pl.kernel,
    mesh=_mesh,
    out_type=jax.ShapeDtypeStruct((ROWS, DPAD), jnp.float32),
    scratch_types=[
        pltpu.VMEM((CH,), jnp.int32),
        pltpu.VMEM((CH,), jnp.int32),
        pltpu.VMEM((CH,), jnp.int32),
        pltpu.VMEM((CH,), jnp.int32),
        pltpu.VMEM((NSLOT, CH, DPAD), jnp.float32),
        pltpu.SemaphoreType.DMA,
        pltpu.SemaphoreType.DMA,
        pltpu.SemaphoreType.DMA,
        pltpu.SemaphoreType.DMA,
        pltpu.SemaphoreType.DMA,
        pltpu.SemaphoreType.DMA,
        pltpu.SemaphoreType.DMA,
        pltpu.SemaphoreType.DMA,
    ],
)
def _gather_kernel(ids_hbm, table_hbm, out_hbm,
                   i0, i1, i2, i3, rows_v,
                   g0, g1, g2, g3, s0, s1, s2, s3):
    idx = (i0, i1, i2, i3)
    sem_g = (g0, g1, g2, g3)
    sem_s = (s0, s1, s2, s3)
    w = lax.axis_index("s") * NC + lax.axis_index("c")
    base = w * RPW

    def load_idx(j, p):
        pltpu.sync_copy(ids_hbm.at[pl.ds(base + j * CH, CH)], idx[p])

    def gather(j, p):
        return pltpu.make_async_copy(
            table_hbm.at[idx[p]], rows_v.at[p], sem_g[p]
        )

    def store(j, p):
        return pltpu.make_async_copy(
            rows_v.at[p], out_hbm.at[pl.ds(base + j * CH, CH)], sem_s[p]
        )

    for p in range(2):
        load_idx(p, p)
        gather(p, p).start()

    def body(t, carry):
        for p in range(NSLOT):
            j = NSLOT * t + p
            gather(j, p).wait()
            store(j, p).start()
            q = (p + 2) % NSLOT
            jn = j + 2

            @pl.when(jn < NCH)
            def _prefetch():
                @pl.when(j >= 2)
                def _drain_store():
                    store(j - 2, q).wait()

                load_idx(jn, q)
                gather(jn, q).start()

        return carry

    lax.fori_loop(0, NCH // NSLOT, body, 0)

    store(NCH - 2, (NCH - 2) % NSLOT).wait()
    store(NCH - 1, (NCH - 1) % NSLOT).wait()


BB = 8


def _depad_body(in_ref, out_ref):
    out_ref[...] = in_ref[...].reshape(BB, SEQ, DPAD)[:, :, :DIM]


_depad_kernel = pl.pallas_call(
    _depad_body,
    grid=(BATCH // BB,),
    in_specs=[pl.BlockSpec((BB * SEQ, DPAD), lambda i: (i, 0))],
    out_specs=pl.BlockSpec((BB, SEQ, DIM), lambda i: (i, 0, 0)),
    out_shape=jax.ShapeDtypeStruct((BATCH, SEQ, DIM), jnp.float32),
)


def kernel(input_ids, table):
    ids_t = jnp.transpose(input_ids, (1, 0)).reshape(ROWS)
    table_pad = jnp.pad(table, ((0, 0), (0, DPAD - DIM)))
    padded = _gather_kernel(ids_t.astype(jnp.int32), table_pad)
    return _depad_kernel(padded)


# speedup vs baseline: 4.4082x; 2.1042x over previous
import functools

import jax
import jax.numpy as jnp
from jax import lax
from jax.experimental import pallas as pl
from jax.experimental.pallas import tpu as pltpu
from jax.experimental.pallas import tpu_sc as plsc

VOCAB = 100000
DIM = 100
DPAD = 128
SEQ = 200
BATCH = 4096

NC = 2
NS = 16
NW = NC * NS
ROWS = SEQ * BATCH
RPW = ROWS // NW
CH = 128
NCH = RPW // CH
NSLOT = 4

_mesh = plsc.VectorSubcoreMesh(core_axis_name="c", subcore_axis_name="s")


@functools.partial(
    pl.kernel,
    mesh=_mesh,
    out_type=jax.ShapeDtypeStruct((ROWS, DPAD), jnp.float32),
    scratch_types=[
        pltpu.VMEM((CH,), jnp.int32),
        pltpu.VMEM((CH,), jnp.int32),
        pltpu.VMEM((CH,), jnp.int32),
        pltpu.VMEM((CH,), jnp.int32),
        pltpu.VMEM((NSLOT, CH, DPAD), jnp.float32),
        pltpu.SemaphoreType.DMA,
        pltpu.SemaphoreType.DMA,
        pltpu.SemaphoreType.DMA,
        pltpu.SemaphoreType.DMA,
        pltpu.SemaphoreType.DMA,
        pltpu.SemaphoreType.DMA,
        pltpu.SemaphoreType.DMA,
        pltpu.SemaphoreType.DMA,
    ],
)
def _gather_kernel(ids_hbm, table_hbm, out_hbm,
                   i0, i1, i2, i3, rows_v,
                   g0, g1, g2, g3, s0, s1, s2, s3):
    idx = (i0, i1, i2, i3)
    sem_g = (g0, g1, g2, g3)
    sem_s = (s0, s1, s2, s3)
    w = lax.axis_index("s") * NC + lax.axis_index("c")
    base = w * RPW

    def load_idx(j, p):
        pltpu.sync_copy(ids_hbm.at[pl.ds(base + j * CH, CH)], idx[p])

    def gather(j, p):
        return pltpu.make_async_copy(
            table_hbm.at[idx[p]], rows_v.at[p], sem_g[p]
        )

    def store(j, p):
        return pltpu.make_async_copy(
            rows_v.at[p], out_hbm.at[pl.ds(base + j * CH, CH)], sem_s[p]
        )

    for p in range(2):
        load_idx(p, p)
        gather(p, p).start()

    def body(t, carry):
        for p in range(NSLOT):
            j = NSLOT * t + p
            gather(j, p).wait()
            store(j, p).start()
            q = (p + 2) % NSLOT
            jn = j + 2

            @pl.when(jn < NCH)
            def _prefetch():
                @pl.when(j >= 2)
                def _drain_store():
                    store(j - 2, q).wait()

                load_idx(jn, q)
                gather(jn, q).start()

        return carry

    lax.fori_loop(0, NCH // NSLOT, body, 0)

    store(NCH - 2, (NCH - 2) % NSLOT).wait()
    store(NCH - 1, (NCH - 1) % NSLOT).wait()


def kernel(input_ids, table):
    ids_t = jnp.transpose(input_ids, (1, 0)).reshape(ROWS)
    eye_pad = jnp.eye(DIM, DPAD, dtype=jnp.float32)
    table_pad = jnp.dot(table, eye_pad)
    padded = _gather_kernel(ids_t.astype(jnp.int32), table_pad)
    return padded.reshape(BATCH, SEQ, DPAD)[:, :, :DIM]
